# Initial kernel scaffold; baseline (speedup 1.0000x reference)
import jax
import jax.numpy as jnp
from jax.experimental import pallas as pl


def kernel(lidar_x, lidar_pos, lidar_logvar, edge_lidar_spatial, radar1_x, radar1_pos, radar1_logvar, radar1_batch, edge_radar1_temporal, edge_r1l_src, edge_r1l_dst, radar2_x, radar2_pos, radar2_logvar, radar2_batch, edge_radar2_temporal, edge_r2l_src, edge_r2l_dst, gt_pos, gt_batch, dt):
    raise NotImplementedError("write your pallas kernel here")


# speedup vs baseline: 21.5637x; 21.5637x over previous
import functools
import math

import jax
import jax.numpy as jnp
from jax import lax
from jax.experimental import pallas as pl
from jax.experimental.pallas import tpu as pltpu, tpu_sc as plsc

SCALE_POSE = 10.0
SCALE_RADAR_V = 5.0
L_MIN = 2 * math.log(0.03 / SCALE_POSE + 1e-09)
L_MAX = 2 * math.log(0.2 / SCALE_POSE + 1e-09)
R_MIN = 2 * math.log(0.1 / SCALE_RADAR_V + 1e-09)
R_MAX = 2 * math.log(3.0 / SCALE_RADAR_V + 1e-09)
GHOST = 2.0
W_L_INT = 1.0
W_R_SPAT = 0.1

NC, NS, NW, LN = 2, 16, 32, 16

N_L = 100000
R_L = 100352
N_R = 20000
R_R = 20480
C = 128
K = 16

KL = 8
RT_L = 400
E_PAD_L = NW * RT_L * C
GL = RT_L // KL

KR = 8
RT_R = 80
E_PAD_R = NW * RT_R * C
GR = RT_R // KR


def _pad_edges(idx, e_pad, trash_base, trash_n):
  e = idx.shape[0]
  pad = trash_base + (jnp.arange(e_pad - e, dtype=jnp.int32) % trash_n)
  return jnp.concatenate([idx.astype(jnp.int32), pad])


def _iota16():
  return lax.iota(jnp.int32, LN)


def _col(k):
  return jnp.full((LN,), k, jnp.int32)


def _rsqrt_nr(s2):
  ib = lax.bitcast_convert_type(s2, jnp.int32)
  ih = jnp.int32(0x5F3759DF) - lax.shift_right_logical(ib, 1)
  y = lax.bitcast_convert_type(ih, jnp.float32)
  for _ in range(3):
    y = y * (1.5 - 0.5 * s2 * y * y)
  return y


@functools.lru_cache(maxsize=None)
def _sc_lidar_kernel():
  mesh = plsc.VectorSubcoreMesh(core_axis_name="c", subcore_axis_name="s")

  @functools.partial(
      pl.kernel,
      mesh=mesh,
      out_type=jax.ShapeDtypeStruct((NC, R_L, 8), jnp.float32),
      scratch_types=[
          pltpu.VMEM((KL, C), jnp.int32),
          pltpu.VMEM((KL, C), jnp.int32),
          pltpu.VMEM((KL * C, 8), jnp.float32),
          pltpu.VMEM_SHARED((R_L, 8), jnp.float32),
          pltpu.SemaphoreType.DMA,
          pltpu.SemaphoreType.DMA,
      ],
      compiler_params=pltpu.CompilerParams(use_tc_tiling_on_sc=False),
  )
  def body(tab_h, src_h, dst_h, z_h, out_h, sidx, didx, rows, acc, sem_g,
           sem_s):
    c = lax.axis_index("c")
    s = lax.axis_index("s")
    rpt = R_L // NS
    pltpu.sync_copy(z_h.at[pl.ds(s * rpt, rpt)], acc.at[pl.ds(s * rpt, rpt)])
    plsc.subcore_barrier()
    tile_row0 = (c * NS + s) * RT_L

    def grp(g, carry):
      r0 = tile_row0 + g * KL
      pltpu.sync_copy(src_h.at[pl.ds(r0, KL)], sidx)
      pltpu.sync_copy(dst_h.at[pl.ds(r0, KL)], didx)
      descs = [
          pltpu.async_copy(tab_h.at[sidx.at[j]],
                           rows.at[pl.ds(j * C, C)], sem_g)
          for j in range(KL)
      ]
      for d in descs:
        d.wait()
      descs = [
          pltpu.async_copy(rows.at[pl.ds(j * C, C)], acc.at[didx.at[j]],
                           sem_s, add=True)
          for j in range(KL)
      ]
      for d in descs:
        d.wait()
      return carry

    lax.fori_loop(0, GL, grp, 0)
    plsc.subcore_barrier()
    pltpu.sync_copy(acc.at[pl.ds(s * rpt, rpt)],
                    out_h.at[c, pl.ds(s * rpt, rpt)])

  return body


@functools.lru_cache(maxsize=None)
def _sc_radar_kernel():
  mesh = plsc.VectorSubcoreMesh(core_axis_name="c", subcore_axis_name="s")
  B = KR * C

  @functools.partial(
      pl.kernel,
      mesh=mesh,
      out_type=jax.ShapeDtypeStruct((NC, 6 * R_R), jnp.float32),
      scratch_types=[
          pltpu.VMEM((KR, C), jnp.int32),
          pltpu.VMEM((KR, C), jnp.int32),
          pltpu.VMEM((KR, C), jnp.int32),
          pltpu.VMEM((KR, C), jnp.int32),
          pltpu.VMEM((B,), jnp.float32),
          pltpu.VMEM((B,), jnp.float32),
          pltpu.VMEM((B,), jnp.float32),
          pltpu.VMEM((B,), jnp.float32),
          pltpu.VMEM((B,), jnp.float32),
          pltpu.VMEM((B,), jnp.float32),
          pltpu.VMEM((B,), jnp.float32),
          pltpu.VMEM((B,), jnp.float32),
          pltpu.VMEM((B,), jnp.float32),
          pltpu.VMEM((B,), jnp.float32),
          pltpu.VMEM((B,), jnp.float32),
          pltpu.VMEM_SHARED((R_R,), jnp.float32),
          pltpu.VMEM_SHARED((R_R,), jnp.float32),
          pltpu.VMEM_SHARED((R_R,), jnp.float32),
          pltpu.VMEM_SHARED((R_L,), jnp.float32),
          pltpu.VMEM_SHARED((R_L,), jnp.float32),
          pltpu.VMEM_SHARED((R_L,), jnp.float32),
          pltpu.VMEM_SHARED((R_R,), jnp.float32),
          pltpu.VMEM_SHARED((R_R,), jnp.float32),
          pltpu.VMEM_SHARED((R_R,), jnp.float32),
          pltpu.VMEM_SHARED((R_R,), jnp.float32),
          pltpu.VMEM_SHARED((R_R,), jnp.float32),
          pltpu.VMEM_SHARED((R_R,), jnp.float32),
          pltpu.SemaphoreType.DMA,
          pltpu.SemaphoreType.DMA,
      ],
      compiler_params=pltpu.CompilerParams(use_tc_tiling_on_sc=False),
  )
  def body(pxh, pyh, pzh, lxh, lyh, lzh, ts_h, td_h, rs_h, rd_h,
           z_h, out_h, tsb, tdb, rsb, rdb, sxb, syb, szb,
           dxb, dyb, dzb, uxb, uyb, uzb, onesb, d2b, pxs, pys, pzs, lxs,
           lys, lzs, accx, accy, accz, acct, accd, accc, sem_g, sem_s):
    c = lax.axis_index("c")
    s = lax.axis_index("s")
    rpt = R_R // NS
    rptl = R_L // NS
    sl_r = pl.ds(s * rpt, rpt)
    sl_l = pl.ds(s * rptl, rptl)
    for a in (accx, accy, accz, acct, accd, accc):
      pltpu.sync_copy(z_h.at[sl_r], a.at[sl_r])
    pltpu.sync_copy(pxh.at[sl_r], pxs.at[sl_r])
    pltpu.sync_copy(pyh.at[sl_r], pys.at[sl_r])
    pltpu.sync_copy(pzh.at[sl_r], pzs.at[sl_r])
    pltpu.sync_copy(lxh.at[sl_l], lxs.at[sl_l])
    pltpu.sync_copy(lyh.at[sl_l], lys.at[sl_l])
    pltpu.sync_copy(lzh.at[sl_l], lzs.at[sl_l])

    one16 = jnp.full((LN,), 1.0, jnp.float32)

    def prefill(i, carry):
      onesb[pl.ds(i * LN, LN)] = one16
      return carry

    lax.fori_loop(0, B // LN, prefill, 0)
    plsc.subcore_barrier()

    tile_row0 = (c * NS + s) * RT_R

    def grp_t(g, carry):
      r0 = tile_row0 + g * KR
      pltpu.sync_copy(ts_h.at[pl.ds(r0, KR)], tsb)
      pltpu.sync_copy(td_h.at[pl.ds(r0, KR)], tdb)

      def gat(q, carry2):
        sl = pl.ds(q * C, C)
        d1 = pltpu.async_copy(pxs.at[tsb.at[q]], sxb.at[sl], sem_g)
        d2 = pltpu.async_copy(pys.at[tsb.at[q]], syb.at[sl], sem_g)
        d3 = pltpu.async_copy(pzs.at[tsb.at[q]], szb.at[sl], sem_g)
        d4 = pltpu.async_copy(pxs.at[tdb.at[q]], dxb.at[sl], sem_g)
        d5 = pltpu.async_copy(pys.at[tdb.at[q]], dyb.at[sl], sem_g)
        d6 = pltpu.async_copy(pzs.at[tdb.at[q]], dzb.at[sl], sem_g)
        for d in (d1, d2, d3, d4, d5, d6):
          d.wait()
        return carry2

      lax.fori_loop(0, KR, gat, 0)

      def step(i, carry2):
        sl = pl.ds(i * LN, LN)
        mx = dxb[sl] - sxb[sl]
        my = dyb[sl] - syb[sl]
        mz = dzb[sl] - szb[sl]
        s2 = mx * mx + my * my + mz * mz + 1e-18
        den = s2 * _rsqrt_nr(s2) + 1e-09
        uxb[sl] = mx / den
        uyb[sl] = my / den
        uzb[sl] = mz / den
        return carry2

      lax.fori_loop(0, B // LN, step, 0)
      descs = []
      for j in range(KR):
        sl = pl.ds(j * C, C)
        descs.append(pltpu.async_copy(uxb.at[sl], accx.at[tdb.at[j]], sem_s,
                                      add=True))
        descs.append(pltpu.async_copy(uyb.at[sl], accy.at[tdb.at[j]], sem_s,
                                      add=True))
        descs.append(pltpu.async_copy(uzb.at[sl], accz.at[tdb.at[j]], sem_s,
                                      add=True))
        descs.append(pltpu.async_copy(onesb.at[sl], acct.at[tdb.at[j]], sem_s,
                                      add=True))
      for d in descs:
        d.wait()
      return carry

    lax.fori_loop(0, GR, grp_t, 0)

    def grp_s(g, carry):
      r0 = tile_row0 + g * KR
      pltpu.sync_copy(rs_h.at[pl.ds(r0, KR)], rsb)
      pltpu.sync_copy(rd_h.at[pl.ds(r0, KR)], rdb)

      def gat(q, carry2):
        sl = pl.ds(q * C, C)
        d1 = pltpu.async_copy(pxs.at[rsb.at[q]], sxb.at[sl], sem_g)
        d2 = pltpu.async_copy(pys.at[rsb.at[q]], syb.at[sl], sem_g)
        d3 = pltpu.async_copy(pzs.at[rsb.at[q]], szb.at[sl], sem_g)
        d4 = pltpu.async_copy(lxs.at[rdb.at[q]], dxb.at[sl], sem_g)
        d5 = pltpu.async_copy(lys.at[rdb.at[q]], dyb.at[sl], sem_g)
        d6 = pltpu.async_copy(lzs.at[rdb.at[q]], dzb.at[sl], sem_g)
        for d in (d1, d2, d3, d4, d5, d6):
          d.wait()
        return carry2

      lax.fori_loop(0, KR, gat, 0)

      def step(i, carry2):
        sl = pl.ds(i * LN, LN)
        dx = sxb[sl] - dxb[sl]
        dy = syb[sl] - dyb[sl]
        dz = szb[sl] - dzb[sl]
        d2b[sl] = dx * dx + dy * dy + dz * dz
        return carry2

      lax.fori_loop(0, B // LN, step, 0)
      descs = []
      for j in range(KR):
        sl = pl.ds(j * C, C)
        descs.append(pltpu.async_copy(d2b.at[sl], accd.at[rsb.at[j]], sem_s,
                                      add=True))
        descs.append(pltpu.async_copy(onesb.at[sl], accc.at[rsb.at[j]], sem_s,
                                      add=True))
      for d in descs:
        d.wait()
      return carry

    lax.fori_loop(0, GR, grp_s, 0)
    plsc.subcore_barrier()
    for col, a in enumerate((accx, accy, accz, acct, accd, accc)):
      pltpu.sync_copy(a.at[sl_r],
                      out_h.at[c, pl.ds(col * R_R + s * rpt, rpt)])

  return body


def _tc_lidar_finalize(acc, node):
  blk = 2048
  grid = R_L // blk

  def body(acc_ref, node_ref, o_ref):
    i = pl.program_id(0)
    a = acc_ref[0] + acc_ref[1]
    cm = jnp.maximum(a[:, 4:5], 1.0)
    mx = a[:, 0:1] / cm
    my = a[:, 1:2] / cm
    mz = a[:, 2:3] / cm
    mi = a[:, 3:4] / cm
    x = node_ref[:, 0:1]
    y = node_ref[:, 1:2]
    z = node_ref[:, 2:3]
    it = node_ref[:, 3:4]
    lv = jnp.clip(node_ref[:, 4:5], L_MIN, L_MAX)
    vld = node_ref[:, 5:6]
    sres = (x - mx) ** 2 + (y - my) ** 2 + (z - mz) ** 2
    comb = sres + W_L_INT * (it - mi) ** 2
    nll = 0.5 * jnp.exp(-lv) * comb + 0.5 * lv

    @pl.when(i == 0)
    def _():
      o_ref[...] = jnp.zeros((1, 1), jnp.float32)

    o_ref[...] += jnp.reshape(jnp.sum(nll * vld), (1, 1))

  return pl.pallas_call(
      body,
      grid=(grid,),
      in_specs=[
          pl.BlockSpec((NC, blk, 8), lambda i: (0, i, 0)),
          pl.BlockSpec((blk, 8), lambda i: (i, 0)),
      ],
      out_specs=pl.BlockSpec((1, 1), lambda i: (0, 0)),
      out_shape=jax.ShapeDtypeStruct((1, 1), jnp.float32),
  )(acc, node)


def _tc_radar_finalize(acc, node, gtm, sdt):
  blk = 1024
  grid = R_R // blk

  def body(acc_ref, node_ref, gt_ref, sdt_ref, ot_ref, os_ref, or_ref):
    i = pl.program_id(0)
    a = acc_ref[0] + acc_ref[1]
    dt = sdt_ref[0, 0]
    cm = jnp.maximum(a[3:4, :], 1.0)
    mx = a[0:1, :] / cm
    my = a[1:2, :] / cm
    mz = a[2:3, :] / cm
    nrm = jnp.sqrt(mx * mx + my * my + mz * mz + 1e-18) + 1e-09
    spd = jnp.abs(node_ref[3:4, :])
    ppx = node_ref[0:1, :] + spd * (mx / nrm) * dt
    ppy = node_ref[1:2, :] + spd * (my / nrm) * dt
    ppz = node_ref[2:3, :] + spd * (mz / nrm) * dt
    gx = gt_ref[:, 0:1]
    gy = gt_ref[:, 1:2]
    gz = gt_ref[:, 2:3]
    gb = gt_ref[:, 3:4]
    d2 = (ppx - gx) ** 2 + (ppy - gy) ** 2 + (ppz - gz) ** 2
    same = node_ref[5:6, :] == gb
    mind = jnp.min(jnp.where(same, d2, 1e30), axis=0, keepdims=True)
    phys = jnp.where(mind < 1e29, mind, 0.0)
    lvr = jnp.clip(node_ref[4:5, :], R_MIN, R_MAX)
    den = 2.0 * jnp.exp(lvr) * dt * dt + 1e-09
    sd = a[4:5, :]
    cn = a[5:6, :]
    spat = jnp.where(cn > 0, sd / jnp.maximum(cn, 1.0) ** 2, GHOST)
    vld = node_ref[6:7, :]

    @pl.when(i == 0)
    def _():
      ot_ref[...] = jnp.zeros((1, 1), jnp.float32)
      os_ref[...] = jnp.zeros((1, 1), jnp.float32)
      or_ref[...] = jnp.zeros((1, 1), jnp.float32)

    ot_ref[...] += jnp.reshape(jnp.sum(phys / den * vld), (1, 1))
    os_ref[...] += jnp.reshape(jnp.sum(spat / den * vld), (1, 1))
    or_ref[...] += jnp.reshape(jnp.sum(0.5 * lvr * vld), (1, 1))

  return pl.pallas_call(
      body,
      grid=(grid,),
      in_specs=[
          pl.BlockSpec((NC, 6, blk), lambda i: (0, 0, i)),
          pl.BlockSpec((8, blk), lambda i: (0, i)),
          pl.BlockSpec((512, 8), lambda i: (0, 0)),
          pl.BlockSpec(memory_space=pltpu.SMEM),
      ],
      out_specs=[pl.BlockSpec((1, 1), lambda i: (0, 0))] * 3,
      out_shape=[jax.ShapeDtypeStruct((1, 1), jnp.float32)] * 3,
  )(acc, node, gtm, sdt)


def _radar_term(pos, x, logvar, batch, e_temp, e_rl_src, e_rl_dst, lcols,
                z_r, gtm, sdt):
  padx = jnp.zeros((R_R,), jnp.float32).at[:N_R].set(pos[:, 0])
  pady = jnp.zeros((R_R,), jnp.float32).at[:N_R].set(pos[:, 1])
  padz = jnp.zeros((R_R,), jnp.float32).at[:N_R].set(pos[:, 2])

  ts = _pad_edges(e_temp[0], E_PAD_R, N_R, R_R - N_R)
  td = _pad_edges(e_temp[1], E_PAD_R, N_R, R_R - N_R)
  rs = _pad_edges(e_rl_src, E_PAD_R, N_R, R_R - N_R)
  rd = _pad_edges(e_rl_dst, E_PAD_R, N_L, R_L - N_L)

  acc = _sc_radar_kernel()(
      padx, pady, padz, lcols[0], lcols[1], lcols[2],
      ts.reshape(E_PAD_R // C, C), td.reshape(E_PAD_R // C, C),
      rs.reshape(E_PAD_R // C, C), rd.reshape(E_PAD_R // C, C), z_r)
  acc = acc.reshape(NC, 6, R_R)

  node = jnp.zeros((8, R_R), jnp.float32)
  node = node.at[0:3, :N_R].set(pos.T)
  node = node.at[3, :N_R].set(x[:, 2])
  node = node.at[4, :N_R].set(logvar[:, 0])
  node = node.at[5, :N_R].set(batch.astype(jnp.float32))
  node = node.at[6, :N_R].set(1.0)

  o_t, o_s, o_r = _tc_radar_finalize(acc, node, gtm, sdt)
  return (o_t[0, 0] / N_R + W_R_SPAT * (o_s[0, 0] / N_R)
          + o_r[0, 0] / N_R)


def kernel(lidar_x, lidar_pos, lidar_logvar, edge_lidar_spatial, radar1_x,
           radar1_pos, radar1_logvar, radar1_batch, edge_radar1_temporal,
           edge_r1l_src, edge_r1l_dst, radar2_x, radar2_pos, radar2_logvar,
           radar2_batch, edge_radar2_temporal, edge_r2l_src, edge_r2l_dst,
           gt_pos, gt_batch, dt):
  dt_f = jnp.asarray(dt, jnp.float32)
  safe_dt = jnp.where(dt_f > 0.01, dt_f, 0.1)
  sdt = jnp.reshape(safe_dt, (1, 1))

  ltab = jnp.zeros((R_L, 8), jnp.float32)
  ltab = ltab.at[:N_L, 0:3].set(lidar_pos)
  ltab = ltab.at[:N_L, 3].set(lidar_x[:, 2])
  ltab = ltab.at[:N_L, 4].set(1.0)

  lsrc = _pad_edges(edge_lidar_spatial[0], E_PAD_L, N_L, R_L - N_L)
  ldst = _pad_edges(edge_lidar_spatial[1], E_PAD_L, N_L, R_L - N_L)

  z_l = jnp.zeros((R_L, 8), jnp.float32)
  z_r = jnp.zeros((R_R,), jnp.float32)
  lcols = (jnp.zeros((R_L,), jnp.float32).at[:N_L].set(lidar_pos[:, 0]),
           jnp.zeros((R_L,), jnp.float32).at[:N_L].set(lidar_pos[:, 1]),
           jnp.zeros((R_L,), jnp.float32).at[:N_L].set(lidar_pos[:, 2]))

  acc_l = _sc_lidar_kernel()(
      ltab, lsrc.reshape(E_PAD_L // C, C), ldst.reshape(E_PAD_L // C, C), z_l)

  node_l = jnp.zeros((R_L, 8), jnp.float32)
  node_l = node_l.at[:N_L, 0:3].set(lidar_pos)
  node_l = node_l.at[:N_L, 3].set(lidar_x[:, 2])
  node_l = node_l.at[:N_L, 4].set(lidar_logvar[:, 0])
  node_l = node_l.at[:N_L, 5].set(1.0)

  l_sum = _tc_lidar_finalize(acc_l, node_l)
  total = l_sum[0, 0] / N_L

  gtm = jnp.zeros((512, 8), jnp.float32)
  gtm = gtm.at[:, 0:3].set(gt_pos)
  gtm = gtm.at[:, 3].set(gt_batch.astype(jnp.float32))

  total = total + _radar_term(radar1_pos, radar1_x, radar1_logvar,
                              radar1_batch, edge_radar1_temporal, edge_r1l_src,
                              edge_r1l_dst, lcols, z_r, gtm, sdt)
  total = total + _radar_term(radar2_pos, radar2_x, radar2_logvar,
                              radar2_batch, edge_radar2_temporal, edge_r2l_src,
                              edge_r2l_dst, lcols, z_r, gtm, sdt)
  return total
